# Initial kernel scaffold; baseline (speedup 1.0000x reference)
import jax
import jax.numpy as jnp
from jax.experimental import pallas as pl


def kernel(actions, embedding_table):
    raise NotImplementedError("write your pallas kernel here")


# speedup vs baseline: 1.0949x; 1.0949x over previous
import functools

import jax
import jax.numpy as jnp
from jax import lax
from jax.experimental import pallas as pl
from jax.experimental.pallas import tpu as pltpu
from jax.experimental.pallas import tpu_sc as plsc

_BATCH = 16384
_HIST = 50
_HIDDEN = 32
_B = _BATCH * _HIST
_NC, _NS = 2, 16
_NW = _NC * _NS
_B_PER_W = _B // _NW
_GATHER = 128
_SUPER = 1024
_G_PER_S = _SUPER // _GATHER
_N_SUPER = _B_PER_W // _SUPER

_mesh = plsc.VectorSubcoreMesh(core_axis_name="c", subcore_axis_name="s")


@functools.partial(
    pl.kernel,
    mesh=_mesh,
    out_type=jax.ShapeDtypeStruct((_B, _HIDDEN), jnp.float32),
    compiler_params=pltpu.CompilerParams(use_tc_tiling_on_sc=False),
    scratch_types=[
        pltpu.VMEM((_G_PER_S, _GATHER), jnp.int32),
        pltpu.VMEM((_SUPER, _HIDDEN), jnp.float32),
        pltpu.SemaphoreType.DMA,
    ],
)
def _embed(actions_hbm, table_hbm, out_hbm, idx_v, rows_v, gsem):
    wid = lax.axis_index("s") * _NC + lax.axis_index("c")
    row0 = wid * (_B_PER_W // _GATHER)

    def super_body(i, carry):
        g0 = row0 + i * _G_PER_S
        pltpu.sync_copy(actions_hbm.at[pl.ds(g0, _G_PER_S)], idx_v)
        copies = []
        for j in range(_G_PER_S):
            copies.append(
                pltpu.async_copy(
                    table_hbm.at[idx_v.at[j]],
                    rows_v.at[pl.ds(j * _GATHER, _GATHER)],
                    gsem,
                )
            )
        for c in copies:
            c.wait()
        pltpu.sync_copy(rows_v, out_hbm.at[pl.ds(g0 * _GATHER, _SUPER)])
        return carry

    lax.fori_loop(0, _N_SUPER, super_body, 0)


def kernel(actions, embedding_table):
    flat = actions.reshape(_B // _GATHER, _GATHER).astype(jnp.int32)
    out = _embed(flat, embedding_table)
    return out.reshape(_BATCH, _HIST, _HIDDEN)


# speedup vs baseline: 1.1143x; 1.0177x over previous
import functools

import jax
import jax.numpy as jnp
from jax import lax
from jax.experimental import pallas as pl
from jax.experimental.pallas import tpu as pltpu
from jax.experimental.pallas import tpu_sc as plsc

_BATCH = 16384
_HIST = 50
_HIDDEN = 32
_B = _BATCH * _HIST
_NC, _NS = 2, 16
_NW = _NC * _NS
_B_PER_W = _B // _NW
_GATHER = 128
_CHUNK = 1280
_G_PER_C = _CHUNK // _GATHER
_N_CHUNK = _B_PER_W // _CHUNK
_IDX_ROWS = _B_PER_W // _GATHER

_mesh = plsc.VectorSubcoreMesh(core_axis_name="c", subcore_axis_name="s")


@functools.partial(
    pl.kernel,
    mesh=_mesh,
    out_type=jax.ShapeDtypeStruct((_B, _HIDDEN), jnp.float32),
    compiler_params=pltpu.CompilerParams(use_tc_tiling_on_sc=False),
    scratch_types=[
        pltpu.VMEM((_IDX_ROWS, _GATHER), jnp.int32),
        pltpu.VMEM((_CHUNK, _HIDDEN), jnp.float32),
        pltpu.VMEM((_CHUNK, _HIDDEN), jnp.float32),
        pltpu.SemaphoreType.DMA,
        pltpu.SemaphoreType.DMA,
        pltpu.SemaphoreType.DMA,
        pltpu.SemaphoreType.DMA,
    ],
)
def _embed(actions_hbm, table_hbm, out_hbm, idx_all, buf0, buf1,
           gs0, gs1, os0, os1):
    wid = lax.axis_index("s") * _NC + lax.axis_index("c")
    out_base = wid * _B_PER_W

    pltpu.sync_copy(actions_hbm.at[pl.ds(wid * _IDX_ROWS, _IDX_ROWS)], idx_all)

    bufs = (buf0, buf1)
    gsems = (gs0, gs1)
    osems = (os0, os1)

    def fire(i, b):
        for j in range(_G_PER_C):
            pltpu.async_copy(
                table_hbm.at[idx_all.at[i * _G_PER_C + j]],
                bufs[b].at[pl.ds(j * _GATHER, _GATHER)],
                gsems[b],
            )

    def drain_gathers(b):
        pltpu.make_async_copy(
            table_hbm.at[pl.ds(0, _CHUNK)], bufs[b], gsems[b]
        ).wait()

    def store(i, b):
        pltpu.async_copy(
            bufs[b], out_hbm.at[pl.ds(out_base + i * _CHUNK, _CHUNK)], osems[b]
        )

    def wait_store(b):
        pltpu.make_async_copy(
            bufs[b], out_hbm.at[pl.ds(0, _CHUNK)], osems[b]
        ).wait()

    fire(0, 0)
    fire(1, 1)
    drain_gathers(0)
    store(0, 0)

    def step(t, carry):
        i0 = 1 + 2 * t
        for k in range(2):
            i = i0 + k
            b = (1, 0)[k]
            nb = (0, 1)[k]
            wait_store(nb)
            fire(i + 1, nb)
            drain_gathers(b)
            store(i, b)
        return carry

    lax.fori_loop(0, (_N_CHUNK - 2) // 2, step, 0)

    drain_gathers(1)
    store(_N_CHUNK - 1, 1)
    wait_store(0)
    wait_store(1)


def kernel(actions, embedding_table):
    flat = actions.reshape(_B // _GATHER, _GATHER).astype(jnp.int32)
    out = _embed(flat, embedding_table)
    return out.reshape(_BATCH, _HIST, _HIDDEN)


# speedup vs baseline: 1.9354x; 1.7369x over previous
import functools

import jax
import jax.numpy as jnp
from jax import lax
from jax.experimental import pallas as pl
from jax.experimental.pallas import tpu as pltpu
from jax.experimental.pallas import tpu_sc as plsc

_BATCH = 16384
_HIST = 50
_HIDDEN = 32
_B = _BATCH * _HIST
_NC, _NS = 2, 16
_NW = _NC * _NS
_BT = _BATCH // 128
_BT_PER_W = _BT // _NW
_G = 5
_NGRP = _HIST // _G

_mesh = plsc.VectorSubcoreMesh(core_axis_name="c", subcore_axis_name="s")


@functools.partial(
    pl.kernel,
    mesh=_mesh,
    out_type=jax.ShapeDtypeStruct((_B, _HIDDEN), jnp.float32),
    compiler_params=pltpu.CompilerParams(use_tc_tiling_on_sc=False),
    scratch_types=[
        pltpu.VMEM((_HIST, 128), jnp.int32),
        pltpu.VMEM((_G * 128, _HIDDEN), jnp.float32),
        pltpu.VMEM((_G * 128, _HIDDEN), jnp.float32),
        pltpu.VMEM((_G * 128, _HIDDEN), jnp.float32),
        pltpu.SemaphoreType.DMA,
        pltpu.SemaphoreType.DMA,
        pltpu.SemaphoreType.DMA,
        pltpu.SemaphoreType.DMA,
        pltpu.SemaphoreType.DMA,
        pltpu.SemaphoreType.DMA,
    ],
)
def _embed(actions_hbm, table_hbm, out_hbm, idx_slab, slot0, slot1, slot2,
           gs0, gs1, gs2, os0, os1, os2):
    wid = lax.axis_index("s") * _NC + lax.axis_index("c")
    slots = (slot0, slot1, slot2)
    gsems = (gs0, gs1, gs2)
    osems = (os0, os1, os2)

    def fire(g, btg, s):
        for j in range(_G):
            h = g * _G + j
            pltpu.async_copy(
                table_hbm.at[idx_slab.at[h]],
                slots[s].at[pl.ds(j * 128, 128)],
                gsems[s],
            )

    def drain_gathers(s):
        pltpu.make_async_copy(
            table_hbm.at[pl.ds(0, _G * 128)], slots[s], gsems[s]
        ).wait()

    def store(g, btg, s):
        for j in range(_G):
            h = g * _G + j
            pltpu.async_copy(
                slots[s].at[pl.ds(j * 128, 128)],
                out_hbm.at[pl.ds(h * _BATCH + btg * 128, 128)],
                osems[s],
            )

    def wait_store(s):
        pltpu.make_async_copy(
            slots[s], out_hbm.at[pl.ds(0, _G * 128)], osems[s]
        ).wait()

    def btile_body(bt, carry):
        btg = wid * _BT_PER_W + bt
        pltpu.sync_copy(
            actions_hbm.at[pl.ds(0, _HIST), pl.ds(btg * 128, 128)], idx_slab)

        fire(0, btg, 0)
        fire(1, btg, 1)
        for g in range(_NGRP):
            s = g % 3
            drain_gathers(s)
            if g + 2 < _NGRP:
                ns = (g + 2) % 3
                if g >= 1:
                    wait_store(ns)
                fire(g + 2, btg, ns)
            store(g, btg, s)
        wait_store(0)
        wait_store(1)
        wait_store(2)
        return carry

    lax.fori_loop(0, _BT_PER_W, btile_body, 0)


def kernel(actions, embedding_table):
    actions_t = jnp.transpose(actions).astype(jnp.int32)
    out2 = _embed(actions_t, embedding_table)
    return out2.reshape(_HIST, _BATCH, _HIDDEN).transpose(1, 0, 2)


# speedup vs baseline: 2.1493x; 1.1105x over previous
import functools

import jax
import jax.numpy as jnp
from jax import lax
from jax.experimental import pallas as pl
from jax.experimental.pallas import tpu as pltpu
from jax.experimental.pallas import tpu_sc as plsc

_BATCH = 16384
_HIST = 50
_HIDDEN = 32
_B = _BATCH * _HIST
_NC, _NS = 2, 16
_NW = _NC * _NS
_BT = _BATCH // 128
_BT_PER_W = _BT // _NW
_G = 5
_NGRP = _HIST // _G

_mesh = plsc.VectorSubcoreMesh(core_axis_name="c", subcore_axis_name="s")


@functools.partial(
    pl.kernel,
    mesh=_mesh,
    out_type=jax.ShapeDtypeStruct((_BATCH, 2048), jnp.float32),
    compiler_params=pltpu.CompilerParams(use_tc_tiling_on_sc=False),
    scratch_types=[
        pltpu.VMEM((_HIST, 128), jnp.int32),
        pltpu.VMEM((_G * 128, _HIDDEN), jnp.float32),
        pltpu.VMEM((_G * 128, _HIDDEN), jnp.float32),
        pltpu.VMEM((_G * 128, _HIDDEN), jnp.float32),
        pltpu.SemaphoreType.DMA,
        pltpu.SemaphoreType.DMA,
        pltpu.SemaphoreType.DMA,
        pltpu.SemaphoreType.DMA,
        pltpu.SemaphoreType.DMA,
        pltpu.SemaphoreType.DMA,
    ],
)
def _embed(actions_hbm, table_hbm, out_hbm, idx_slab, slot0, slot1, slot2,
           gs0, gs1, gs2, os0, os1, os2):
    wid = lax.axis_index("s") * _NC + lax.axis_index("c")
    slots = (slot0, slot1, slot2)
    gsems = (gs0, gs1, gs2)
    osems = (os0, os1, os2)

    def fire(g, btg, s):
        for j in range(_G):
            h = g * _G + j
            pltpu.async_copy(
                table_hbm.at[idx_slab.at[h]],
                slots[s].at[pl.ds(j * 128, 128)],
                gsems[s],
            )

    def drain_gathers(s):
        pltpu.make_async_copy(
            table_hbm.at[pl.ds(0, _G * 128)], slots[s], gsems[s]
        ).wait()

    def store(g, btg, s):
        for j in range(_G):
            h = g * _G + j
            pltpu.async_copy(
                slots[s].at[pl.ds(j * 128, 128)],
                out_hbm.at[pl.ds(btg * 128, 128), pl.ds(h * _HIDDEN, _HIDDEN)],
                osems[s],
            )

    def wait_store(s):
        pltpu.make_async_copy(
            slots[s],
            out_hbm.at[pl.ds(0, _G * 128), pl.ds(0, _HIDDEN)],
            osems[s],
        ).wait()

    def btile_body(bt, carry):
        btg = wid * _BT_PER_W + bt
        pltpu.sync_copy(
            actions_hbm.at[pl.ds(0, _HIST), pl.ds(btg * 128, 128)], idx_slab)

        fire(0, btg, 0)
        fire(1, btg, 1)
        for g in range(_NGRP):
            s = g % 3
            drain_gathers(s)
            if g + 2 < _NGRP:
                ns = (g + 2) % 3
                if g >= 1:
                    wait_store(ns)
                fire(g + 2, btg, ns)
            store(g, btg, s)
        wait_store(0)
        wait_store(1)
        wait_store(2)
        return carry

    lax.fori_loop(0, _BT_PER_W, btile_body, 0)


def kernel(actions, embedding_table):
    actions_t = jnp.transpose(actions).astype(jnp.int32)
    out2 = _embed(actions_t, embedding_table)
    return out2[:, : _HIST * _HIDDEN].reshape(_BATCH, _HIST, _HIDDEN)
